# baseline (speedup 1.0000x reference)
import jax
import jax.numpy as jnp
from jax.experimental import pallas as pl
from jax.experimental.pallas import tpu as pltpu

_TN = 2048


def _fc_kernel(x_ref, w_ref, b_ref, o_ref):
    xb = x_ref[...].astype(jnp.bfloat16)
    wb = w_ref[...].astype(jnp.bfloat16)
    acc = jax.lax.dot_general(
        xb, wb, (((1,), (1,)), ((), ())),
        preferred_element_type=jnp.float32,
    )
    o_ref[...] = acc + b_ref[...]


def kernel(x, W, b):
    batch, in_dim = x.shape
    out_dim = W.shape[0]
    b2 = b.reshape(1, out_dim)
    return pl.pallas_call(
        _fc_kernel,
        grid=(pl.cdiv(out_dim, _TN),),
        in_specs=[
            pl.BlockSpec((batch, in_dim), lambda j: (0, 0)),
            pl.BlockSpec((_TN, in_dim), lambda j: (j, 0)),
            pl.BlockSpec((1, _TN), lambda j: (0, j)),
        ],
        out_specs=pl.BlockSpec((batch, _TN), lambda j: (0, j)),
        out_shape=jax.ShapeDtypeStruct((batch, out_dim), jnp.float32),
        compiler_params=pltpu.CompilerParams(
            dimension_semantics=("arbitrary",),
        ),
    )(x, W, b2)


# speedup vs baseline: 1.0059x; 1.0059x over previous
import jax
import jax.numpy as jnp
from jax.experimental import pallas as pl
from jax.experimental.pallas import tpu as pltpu

_TN = 4096


def _fc_kernel(x_ref, w_ref, b_ref, o_ref):
    xb = x_ref[...].astype(jnp.bfloat16)
    wb = w_ref[...].astype(jnp.bfloat16)
    acc = jax.lax.dot_general(
        xb, wb, (((1,), (1,)), ((), ())),
        preferred_element_type=jnp.float32,
    )
    o_ref[...] = acc + b_ref[...]


def kernel(x, W, b):
    batch, in_dim = x.shape
    out_dim = W.shape[0]
    b2 = b.reshape(1, out_dim)
    return pl.pallas_call(
        _fc_kernel,
        grid=(pl.cdiv(out_dim, _TN),),
        in_specs=[
            pl.BlockSpec((batch, in_dim), lambda j: (0, 0)),
            pl.BlockSpec((_TN, in_dim), lambda j: (j, 0)),
            pl.BlockSpec((1, _TN), lambda j: (0, j)),
        ],
        out_specs=pl.BlockSpec((batch, _TN), lambda j: (0, j)),
        out_shape=jax.ShapeDtypeStruct((batch, out_dim), jnp.float32),
        compiler_params=pltpu.CompilerParams(
            dimension_semantics=("arbitrary",),
        ),
    )(x, W, b2)
